# baseline (device time: 14251 ns/iter reference)
import jax
import jax.numpy as jnp
from jax import lax
from jax.experimental import pallas as pl
from jax.experimental.pallas import tpu as pltpu

N_DEV = 16
N_OUT_CHUNK = 4


def kernel(x):
    m_rows, n_cols = x.shape
    rpc = m_rows // N_OUT_CHUNK

    def body(x_hbm, out_hbm, xv, ev, my_stats, gbuf,
             in_sem, out_sems, send_sems, recv_sems):
        me = lax.axis_index("i")

        bsem = pltpu.get_barrier_semaphore()
        for p in range(N_DEV):
            pl.semaphore_signal(
                bsem, inc=1,
                device_id=(p,), device_id_type=pl.DeviceIdType.MESH,
            )

        fetch = pltpu.make_async_copy(x_hbm, xv, in_sem)
        fetch.start()
        fetch.wait()

        xl = xv[...]
        m_loc = jnp.max(xl, axis=1, keepdims=True)
        e = jnp.exp(xl - m_loc)
        ev[...] = e
        s_loc = jnp.sum(e, axis=1, keepdims=True)
        my_stats[...] = jnp.concatenate([m_loc, s_loc], axis=1).T

        pl.semaphore_wait(bsem, N_DEV)

        for p in range(N_DEV):
            pltpu.make_async_remote_copy(
                src_ref=my_stats,
                dst_ref=gbuf.at[me],
                send_sem=send_sems.at[p],
                recv_sem=recv_sems.at[me],
                device_id=(p,),
                device_id_type=pl.DeviceIdType.MESH,
            ).start()

        for p in range(N_DEV):
            pltpu.make_async_remote_copy(
                src_ref=my_stats,
                dst_ref=gbuf.at[p],
                send_sem=send_sems.at[p],
                recv_sem=recv_sems.at[p],
                device_id=(p,),
                device_id_type=pl.DeviceIdType.MESH,
            ).wait_recv()

        g = gbuf[...]
        m_all = g[:, 0, :]
        s_all = g[:, 1, :]
        m_glob = jnp.max(m_all, axis=0)
        s_glob = jnp.sum(s_all * jnp.exp(m_all - m_glob[None, :]), axis=0)
        scale = jnp.exp(my_stats[0, :] - m_glob) / s_glob
        scale_col = scale[None, :].T

        for c in range(N_OUT_CHUNK):
            rows = slice(c * rpc, (c + 1) * rpc)
            ev[rows, :] = ev[rows, :] * scale_col[rows, :]
            pltpu.make_async_copy(
                ev.at[rows], out_hbm.at[rows], out_sems.at[c]
            ).start()

        for c in range(N_OUT_CHUNK):
            rows = slice(c * rpc, (c + 1) * rpc)
            pltpu.make_async_copy(
                ev.at[rows], out_hbm.at[rows], out_sems.at[c]
            ).wait()

        for p in range(N_DEV):
            pltpu.make_async_remote_copy(
                src_ref=my_stats,
                dst_ref=gbuf.at[me],
                send_sem=send_sems.at[p],
                recv_sem=recv_sems.at[me],
                device_id=(p,),
                device_id_type=pl.DeviceIdType.MESH,
            ).wait_send()

    out_shape = jax.ShapeDtypeStruct((m_rows, n_cols), jnp.float32)
    return pl.pallas_call(
        body,
        out_shape=out_shape,
        in_specs=[pl.BlockSpec(memory_space=pltpu.HBM)],
        out_specs=pl.BlockSpec(memory_space=pltpu.HBM),
        scratch_shapes=[
            pltpu.VMEM((m_rows, n_cols), jnp.float32),
            pltpu.VMEM((m_rows, n_cols), jnp.float32),
            pltpu.VMEM((2, m_rows), jnp.float32),
            pltpu.VMEM((N_DEV, 2, m_rows), jnp.float32),
            pltpu.SemaphoreType.DMA,
            pltpu.SemaphoreType.DMA((N_OUT_CHUNK,)),
            pltpu.SemaphoreType.DMA((N_DEV,)),
            pltpu.SemaphoreType.DMA((N_DEV,)),
        ],
        compiler_params=pltpu.CompilerParams(collective_id=0),
    )(x)


# device time: 13716 ns/iter; 1.0390x vs baseline; 1.0390x over previous
import jax
import jax.numpy as jnp
from jax import lax
from jax.experimental import pallas as pl
from jax.experimental.pallas import tpu as pltpu

N_DEV = 16


def kernel(x):
    m_rows, n_cols = x.shape

    def body(x_hbm, out_ref, xv, ev, my_stats, gbuf,
             in_sem, send_sems, recv_sems):
        me = lax.axis_index("i")

        bsem = pltpu.get_barrier_semaphore()
        for p in range(N_DEV):
            pl.semaphore_signal(
                bsem, inc=1,
                device_id=(p,), device_id_type=pl.DeviceIdType.MESH,
            )

        fetch = pltpu.make_async_copy(x_hbm, xv, in_sem)
        fetch.start()
        fetch.wait()

        xl = xv[...]
        m_loc = jnp.max(xl, axis=1, keepdims=True)
        e = jnp.exp(xl - m_loc)
        ev[...] = e
        s_loc = jnp.sum(e, axis=1, keepdims=True)
        my_stats[...] = jnp.concatenate([m_loc, s_loc], axis=1).T

        pl.semaphore_wait(bsem, N_DEV)

        for p in range(N_DEV):
            pltpu.make_async_remote_copy(
                src_ref=my_stats,
                dst_ref=gbuf.at[me],
                send_sem=send_sems.at[p],
                recv_sem=recv_sems.at[me],
                device_id=(p,),
                device_id_type=pl.DeviceIdType.MESH,
            ).start()

        for p in range(N_DEV):
            pltpu.make_async_remote_copy(
                src_ref=my_stats,
                dst_ref=gbuf.at[p],
                send_sem=send_sems.at[p],
                recv_sem=recv_sems.at[p],
                device_id=(p,),
                device_id_type=pl.DeviceIdType.MESH,
            ).wait_recv()

        g = gbuf[...]
        m_all = g[:, 0, :]
        s_all = g[:, 1, :]
        m_glob = jnp.max(m_all, axis=0)
        s_glob = jnp.sum(s_all * jnp.exp(m_all - m_glob[None, :]), axis=0)
        scale = jnp.exp(my_stats[0, :] - m_glob) / s_glob
        scale_col = scale[None, :].T

        out_ref[...] = ev[...] * scale_col

        for p in range(N_DEV):
            pltpu.make_async_remote_copy(
                src_ref=my_stats,
                dst_ref=gbuf.at[me],
                send_sem=send_sems.at[p],
                recv_sem=recv_sems.at[me],
                device_id=(p,),
                device_id_type=pl.DeviceIdType.MESH,
            ).wait_send()

    out_shape = jax.ShapeDtypeStruct((m_rows, n_cols), jnp.float32)
    return pl.pallas_call(
        body,
        out_shape=out_shape,
        in_specs=[pl.BlockSpec(memory_space=pltpu.HBM)],
        out_specs=pl.BlockSpec(memory_space=pltpu.VMEM),
        scratch_shapes=[
            pltpu.VMEM((m_rows, n_cols), jnp.float32),
            pltpu.VMEM((m_rows, n_cols), jnp.float32),
            pltpu.VMEM((2, m_rows), jnp.float32),
            pltpu.VMEM((N_DEV, 2, m_rows), jnp.float32),
            pltpu.SemaphoreType.DMA,
            pltpu.SemaphoreType.DMA((N_DEV,)),
            pltpu.SemaphoreType.DMA((N_DEV,)),
        ],
        compiler_params=pltpu.CompilerParams(collective_id=0),
    )(x)
